# baseline (device time: 224225 ns/iter reference)
import jax
import jax.numpy as jnp
from jax import lax
from jax.experimental import pallas as pl
from jax.experimental.pallas import tpu as pltpu

N_DEV = 8
B = 512
D = 256
HS = 512

N_HOPS = 42
N_RS = 21


def kernel(x, Win0, Wout0, Win1, Wout1, Win2, Wout2):
    def body(x_ref, win0_ref, wout0_ref, win1_ref, wout1_ref,
             win2_ref, wout2_ref, out_ref,
             xg0, xg1, xg2, sbuf, rbuf, send_sems, recv_sems):
        me = lax.axis_index("i")
        right = lax.rem(me + 1, N_DEV)

        def hop(src, dst, g):
            rdma = pltpu.make_async_remote_copy(
                src_ref=src, dst_ref=dst,
                send_sem=send_sems.at[g], recv_sem=recv_sems.at[g],
                device_id=(right,), device_id_type=pl.DeviceIdType.MESH,
            )
            rdma.start()
            rdma.wait()

        xg0[me] = x_ref[...].astype(jnp.bfloat16)
        for h in range(N_DEV - 1):
            slot = lax.rem(me - h + N_DEV, N_DEV)
            hop(xg0.at[slot], xg0.at[slot], h)

        win_refs = [win0_ref, win1_ref, win2_ref]
        wout_refs = [wout0_ref, wout1_ref, wout2_ref]
        xg_refs = [xg0, xg1, xg2]

        for l in range(3):
            win = win_refs[l][...].astype(jnp.bfloat16)
            wout = wout_refs[l][...].astype(jnp.bfloat16)
            xg = xg_refs[l]

            def pchunk(c):
                xin = xg[c]
                hc = jnp.maximum(
                    jnp.dot(xin, win, preferred_element_type=jnp.float32),
                    0.0,
                ).astype(jnp.bfloat16)
                return jnp.dot(
                    hc, wout, preferred_element_type=jnp.float32
                ).astype(jnp.bfloat16)

            for h in range(N_DEV - 1):
                r = l * (N_DEV - 1) + h
                c = lax.rem(me - 1 - h + 2 * N_DEV, N_DEV)
                val = pchunk(c)
                if h > 0:
                    val = val + rbuf[r - 1]
                sbuf[r] = val
                hop(sbuf.at[r], rbuf.at[r], 7 + 14 * l + h)

            fin = pchunk(me) + rbuf[l * (N_DEV - 1) + (N_DEV - 2)]

            if l < 2:
                nxt = xg_refs[l + 1]
                nxt[me] = fin
                for h in range(N_DEV - 1):
                    slot = lax.rem(me - h + N_DEV, N_DEV)
                    hop(nxt.at[slot], nxt.at[slot], 14 + 14 * l + h)
            else:
                out_ref[...] = fin.astype(jnp.float32)

    return pl.pallas_call(
        body,
        out_shape=jax.ShapeDtypeStruct((B, D), jnp.float32),
        in_specs=[pl.BlockSpec(memory_space=pltpu.VMEM)] * 7,
        out_specs=pl.BlockSpec(memory_space=pltpu.VMEM),
        scratch_shapes=[
            pltpu.VMEM((N_DEV, B, D), jnp.bfloat16),
            pltpu.VMEM((N_DEV, B, D), jnp.bfloat16),
            pltpu.VMEM((N_DEV, B, D), jnp.bfloat16),
            pltpu.VMEM((N_RS, B, D), jnp.bfloat16),
            pltpu.VMEM((N_RS, B, D), jnp.bfloat16),
            pltpu.SemaphoreType.DMA((N_HOPS,)),
            pltpu.SemaphoreType.DMA((N_HOPS,)),
        ],
    )(x, Win0, Wout0, Win1, Wout1, Win2, Wout2)


# device time: 110875 ns/iter; 2.0223x vs baseline; 2.0223x over previous
import jax
import jax.numpy as jnp
from jax import lax
from jax.experimental import pallas as pl
from jax.experimental.pallas import tpu as pltpu

N_DEV = 8
B = 512
D = 256
HS = 512
N_PHASE = 6


def kernel(x, Win0, Wout0, Win1, Wout1, Win2, Wout2):
    def body(x_ref, win0_ref, wout0_ref, win1_ref, wout1_ref,
             win2_ref, wout2_ref, out_ref,
             xg0, xg1, xg2, sbuf, prbuf, send_sems, recv_sems):
        me = lax.axis_index("i")

        def send(src, dst, p, k, dest):
            rdma = pltpu.make_async_remote_copy(
                src_ref=src, dst_ref=dst,
                send_sem=send_sems.at[p, k],
                recv_sem=recv_sems.at[p, me],
                device_id=(dest,), device_id_type=pl.DeviceIdType.MESH,
            )
            rdma.start()
            return rdma

        def wait_recv(p, src, dst):
            rdma = pltpu.make_async_remote_copy(
                src_ref=dst, dst_ref=dst,
                send_sem=send_sems.at[p, 0],
                recv_sem=recv_sems.at[p, src],
                device_id=(me,), device_id_type=pl.DeviceIdType.MESH,
            )
            rdma.wait_recv()

        pending_sends = []

        def broadcast(xg, chunk, p):
            xg[me] = chunk
            for k in range(1, N_DEV):
                dest = lax.rem(me + k, N_DEV)
                pending_sends.append(send(xg.at[me], xg.at[me], p, k, dest))
            for k in range(1, N_DEV):
                src = lax.rem(me - k + N_DEV, N_DEV)
                wait_recv(p, src, xg.at[src])

        def reduce_scatter(xg, l, p, win_ref, wout_ref):
            win = win_ref[...].astype(jnp.bfloat16)
            wout = wout_ref[...].astype(jnp.bfloat16)

            def pchunk(c):
                hc = jnp.maximum(
                    jnp.dot(xg[c], win, preferred_element_type=jnp.float32),
                    0.0,
                ).astype(jnp.bfloat16)
                return jnp.dot(
                    hc, wout, preferred_element_type=jnp.float32
                ).astype(jnp.bfloat16)

            for k in range(1, N_DEV):
                dest = lax.rem(me + k, N_DEV)
                sbuf[l, dest] = pchunk(dest)
                pending_sends.append(
                    send(sbuf.at[l, dest], prbuf.at[l, me], p, k, dest)
                )
            prbuf[l, me] = pchunk(me)
            for k in range(1, N_DEV):
                src = lax.rem(me - k + N_DEV, N_DEV)
                wait_recv(p, src, prbuf.at[l, src])
            total = jnp.sum(prbuf[l].astype(jnp.float32), axis=0)
            return total

        broadcast(xg0, x_ref[...].astype(jnp.bfloat16), 0)
        t0 = reduce_scatter(xg0, 0, 1, win0_ref, wout0_ref)
        broadcast(xg1, t0.astype(jnp.bfloat16), 2)
        t1 = reduce_scatter(xg1, 1, 3, win1_ref, wout1_ref)
        broadcast(xg2, t1.astype(jnp.bfloat16), 4)
        t2 = reduce_scatter(xg2, 2, 5, win2_ref, wout2_ref)
        out_ref[...] = t2

        for rdma in pending_sends:
            rdma.wait_send()

    return pl.pallas_call(
        body,
        out_shape=jax.ShapeDtypeStruct((B, D), jnp.float32),
        in_specs=[pl.BlockSpec(memory_space=pltpu.VMEM)] * 7,
        out_specs=pl.BlockSpec(memory_space=pltpu.VMEM),
        scratch_shapes=[
            pltpu.VMEM((N_DEV, B, D), jnp.bfloat16),
            pltpu.VMEM((N_DEV, B, D), jnp.bfloat16),
            pltpu.VMEM((N_DEV, B, D), jnp.bfloat16),
            pltpu.VMEM((3, N_DEV, B, D), jnp.bfloat16),
            pltpu.VMEM((3, N_DEV, B, D), jnp.bfloat16),
            pltpu.SemaphoreType.DMA((N_PHASE, N_DEV)),
            pltpu.SemaphoreType.DMA((N_PHASE, N_DEV)),
        ],
    )(x, Win0, Wout0, Win1, Wout1, Win2, Wout2)


# device time: 100380 ns/iter; 2.2338x vs baseline; 1.1046x over previous
import jax
import jax.numpy as jnp
from jax import lax
from jax.experimental import pallas as pl
from jax.experimental.pallas import tpu as pltpu

N_DEV = 8
B = 512
D = 256
HS = 512
N_PHASE = 6

T_ORDER = (1, 3, 4, 2, 5, 7, 6)


def kernel(x, Win0, Wout0, Win1, Wout1, Win2, Wout2):
    def body(x_ref, win0_ref, wout0_ref, win1_ref, wout1_ref,
             win2_ref, wout2_ref, out_ref,
             xg0, xg1, xg2, sbuf, prbuf, send_sems, recv_sems):
        me = lax.axis_index("i")

        def send(src, dst, p, dest):
            rdma = pltpu.make_async_remote_copy(
                src_ref=src, dst_ref=dst,
                send_sem=send_sems.at[p, dest],
                recv_sem=recv_sems.at[p, me],
                device_id=(dest,), device_id_type=pl.DeviceIdType.MESH,
            )
            rdma.start()
            return rdma

        def wait_recv(p, src, dst):
            rdma = pltpu.make_async_remote_copy(
                src_ref=dst, dst_ref=dst,
                send_sem=send_sems.at[p, me],
                recv_sem=recv_sems.at[p, src],
                device_id=(me,), device_id_type=pl.DeviceIdType.MESH,
            )
            rdma.wait_recv()

        pending_sends = []

        def broadcast(xg, chunk, p):
            xg[me] = chunk
            for t in T_ORDER:
                dest = jnp.bitwise_xor(me, t)
                pending_sends.append(send(xg.at[me], xg.at[me], p, dest))

        def layer(xg, l, p_ag, p_rs, win_ref, wout_ref):
            win = win_ref[...].astype(jnp.bfloat16)
            wout = wout_ref[...].astype(jnp.bfloat16)

            def pchunk(c):
                hc = jnp.maximum(
                    jnp.dot(xg[c], win, preferred_element_type=jnp.float32),
                    0.0,
                ).astype(jnp.bfloat16)
                return jnp.dot(
                    hc, wout, preferred_element_type=jnp.float32
                ).astype(jnp.bfloat16)

            prbuf[l, me] = pchunk(me)
            for t in T_ORDER:
                src = jnp.bitwise_xor(me, t)
                wait_recv(p_ag, src, xg.at[src])
                sbuf[l, src] = pchunk(src)
                pending_sends.append(
                    send(sbuf.at[l, src], prbuf.at[l, me], p_rs, src)
                )
            for t in T_ORDER:
                src = jnp.bitwise_xor(me, t)
                wait_recv(p_rs, src, prbuf.at[l, src])
            return jnp.sum(prbuf[l].astype(jnp.float32), axis=0)

        broadcast(xg0, x_ref[...].astype(jnp.bfloat16), 0)
        t0 = layer(xg0, 0, 0, 1, win0_ref, wout0_ref)
        broadcast(xg1, t0.astype(jnp.bfloat16), 2)
        t1 = layer(xg1, 1, 2, 3, win1_ref, wout1_ref)
        broadcast(xg2, t1.astype(jnp.bfloat16), 4)
        t2 = layer(xg2, 2, 4, 5, win2_ref, wout2_ref)
        out_ref[...] = t2

        for rdma in pending_sends:
            rdma.wait_send()

    return pl.pallas_call(
        body,
        out_shape=jax.ShapeDtypeStruct((B, D), jnp.float32),
        in_specs=[pl.BlockSpec(memory_space=pltpu.VMEM)] * 7,
        out_specs=pl.BlockSpec(memory_space=pltpu.VMEM),
        scratch_shapes=[
            pltpu.VMEM((N_DEV, B, D), jnp.bfloat16),
            pltpu.VMEM((N_DEV, B, D), jnp.bfloat16),
            pltpu.VMEM((N_DEV, B, D), jnp.bfloat16),
            pltpu.VMEM((3, N_DEV, B, D), jnp.bfloat16),
            pltpu.VMEM((3, N_DEV, B, D), jnp.bfloat16),
            pltpu.SemaphoreType.DMA((N_PHASE, N_DEV)),
            pltpu.SemaphoreType.DMA((N_PHASE, N_DEV)),
        ],
    )(x, Win0, Wout0, Win1, Wout1, Win2, Wout2)


# device time: 61146 ns/iter; 3.6670x vs baseline; 1.6416x over previous
import jax
import jax.numpy as jnp
from jax import lax
from jax.experimental import pallas as pl
from jax.experimental.pallas import tpu as pltpu

N_DEV = 8
B = 512
D = 256
HS = 512
BP = 1024

PLANE_PEERS = (1, 3, 2)


def kernel(x, Win0, Wout0, Win1, Wout1, Win2, Wout2):
    def body(x_ref, win0_ref, wout0_ref, win1_ref, wout1_ref,
             win2_ref, wout2_ref, out_ref,
             winbuf, woutbuf, swin, swout, xin, psend, precv, totbuf,
             wsend_sems, wrecv_sems, msend_sems, mrecv_sems):
        me = lax.axis_index("i")
        myslot = lax.rem(me, 4)
        myrow = me // 4
        partner = jnp.bitwise_xor(me, 4)

        pending = []

        def send(src, dst, send_sem, recv_sem, dest):
            rdma = pltpu.make_async_remote_copy(
                src_ref=src, dst_ref=dst, send_sem=send_sem,
                recv_sem=recv_sem,
                device_id=(dest,), device_id_type=pl.DeviceIdType.MESH,
            )
            rdma.start()
            pending.append(rdma)

        def wait_recv(dst, recv_sem):
            rdma = pltpu.make_async_remote_copy(
                src_ref=dst, dst_ref=dst, send_sem=wsend_sems.at[0, 0, 0],
                recv_sem=recv_sem,
                device_id=(me,), device_id_type=pl.DeviceIdType.MESH,
            )
            rdma.wait_recv()

        win_refs = [win0_ref, win1_ref, win2_ref]
        wout_refs = [wout0_ref, wout1_ref, wout2_ref]

        for l in range(3):
            swin[l] = win_refs[l][...].astype(jnp.bfloat16)
            swout[l] = wout_refs[l][...].astype(jnp.bfloat16)
        for l in range(3):
            for t in PLANE_PEERS:
                dest = jnp.bitwise_xor(me, t)
                dslot = lax.rem(dest, 4)
                send(swin.at[l], winbuf.at[l, myslot],
                     wsend_sems.at[l, 0, dslot], wrecv_sems.at[l, 0, myslot],
                     dest)
                send(swout.at[l], woutbuf.at[l, myslot],
                     wsend_sems.at[l, 1, dslot], wrecv_sems.at[l, 1, myslot],
                     dest)

        xin[0, pl.ds(myrow * B, B), :] = x_ref[...].astype(jnp.bfloat16)
        send(xin.at[0, pl.ds(myrow * B, B), :],
             xin.at[0, pl.ds(myrow * B, B), :],
             msend_sems.at[0], mrecv_sems.at[0], partner)
        wait_recv(xin.at[0, pl.ds((1 - myrow) * B, B), :], mrecv_sems.at[0])

        for l in range(3):
            X = xin[l]

            def contrib(win_s, wout_s):
                h = jnp.maximum(
                    jnp.dot(X, win_s, preferred_element_type=jnp.float32),
                    0.0,
                ).astype(jnp.bfloat16)
                return jnp.dot(h, wout_s, preferred_element_type=jnp.float32)

            acc = contrib(swin[l], swout[l])
            for t in PLANE_PEERS:
                s = lax.rem(jnp.bitwise_xor(me, t), 4)
                wait_recv(winbuf.at[l, s], wrecv_sems.at[l, 0, s])
                wait_recv(woutbuf.at[l, s], wrecv_sems.at[l, 1, s])
                acc = acc + contrib(winbuf[l, s], woutbuf[l, s])

            psend[l] = acc.astype(jnp.bfloat16)
            send(psend.at[l], precv.at[l],
                 msend_sems.at[1 + l], mrecv_sems.at[1 + l], partner)
            wait_recv(precv.at[l], mrecv_sems.at[1 + l])
            tot = acc + precv[l].astype(jnp.float32)

            if l < 2:
                xin[l + 1] = tot.astype(jnp.bfloat16)
            else:
                totbuf[...] = tot
                out_ref[...] = totbuf[pl.ds(myrow * B, B), :]

        for rdma in pending:
            rdma.wait_send()

    return pl.pallas_call(
        body,
        out_shape=jax.ShapeDtypeStruct((B, D), jnp.float32),
        in_specs=[pl.BlockSpec(memory_space=pltpu.VMEM)] * 7,
        out_specs=pl.BlockSpec(memory_space=pltpu.VMEM),
        scratch_shapes=[
            pltpu.VMEM((3, 4, D, HS), jnp.bfloat16),
            pltpu.VMEM((3, 4, HS, D), jnp.bfloat16),
            pltpu.VMEM((3, D, HS), jnp.bfloat16),
            pltpu.VMEM((3, HS, D), jnp.bfloat16),
            pltpu.VMEM((3, BP, D), jnp.bfloat16),
            pltpu.VMEM((3, BP, D), jnp.bfloat16),
            pltpu.VMEM((3, BP, D), jnp.bfloat16),
            pltpu.VMEM((BP, D), jnp.float32),
            pltpu.SemaphoreType.DMA((3, 2, 4)),
            pltpu.SemaphoreType.DMA((3, 2, 4)),
            pltpu.SemaphoreType.DMA((4,)),
            pltpu.SemaphoreType.DMA((4,)),
        ],
    )(x, Win0, Wout0, Win1, Wout1, Win2, Wout2)
